# baseline (speedup 1.0000x reference)
import functools
import math

import jax
import jax.numpy as jnp
from jax import lax
from jax.experimental import pallas as pl
from jax.experimental.pallas import tpu as pltpu
from jax.experimental.pallas import tpu_sc as plsc

NNODE = 10000
NP = 10240
DP = 768
NCHUNK = 6
D2P = 1536
EB = 112
NB = 56
EPT = EB * NB
EP = 16 * EPT
NACC = 10112
ROWS_PT = NACC // 16
UR = 56
NU = EPT // UR
RING = 4
ASC = 3
G = 64
NT = 128
BM = 512
BMP = 1024


def _sc_body(h_flat, src_all, dst_all, zrows, out, src_v, dst_v, gbuf,
             acc, gsem, ssem):
    cc = lax.axis_index("c")
    ss = lax.axis_index("s")
    w = cc * 16 + ss

    pltpu.sync_copy(src_all.at[w], src_v)
    pltpu.sync_copy(dst_all.at[w], dst_v)

    def _zero_my_rows():
        pltpu.sync_copy(zrows, acc.at[pl.ds(ss * ROWS_PT, ROWS_PT)])

    def _gather(b, sel):
        pltpu.async_copy(h_flat.at[src_v.at[pl.ds(b * UR, UR)]],
                         gbuf.at[sel], gsem.at[sel])

    def _gather_wait(b, sel):
        pltpu.make_async_copy(h_flat.at[src_v.at[pl.ds(b * UR, UR)]],
                              gbuf.at[sel], gsem.at[sel]).wait()

    def _scat(b, sel):
        pltpu.async_copy(gbuf.at[sel], acc.at[dst_v.at[b]], ssem.at[sel],
                         add=True)

    def _scat_wait(b, sel):
        pltpu.make_async_copy(gbuf.at[sel], acc.at[dst_v.at[b]],
                              ssem.at[sel]).wait()

    _zero_my_rows()

    for p in range(NCHUNK):
        plsc.subcore_barrier()

        if p > 0:
            def ob(j, carry):
                src_v[pl.ds(j * 16, 16)] = src_v[pl.ds(j * 16, 16)] + NP
                return carry
            lax.fori_loop(0, EPT // 16, ob, 0)

        for i in range(RING):
            _gather(i, i)

        def eb(g, carry):
            for r in range(RING):
                b = g * RING + r
                fs = (r - ASC) % RING

                @pl.when(b >= ASC)
                def _():
                    _scat_wait(b - ASC, fs)

                    @pl.when(b - ASC + RING < NU)
                    def _():
                        _gather(b - ASC + RING, fs)
                _gather_wait(b, r)
                _scat(b, r)
            return carry
        lax.fori_loop(0, NU // RING, eb, 0)
        for k in range(ASC):
            b = NU - ASC + k
            _scat_wait(b, b % RING)

        plsc.subcore_barrier()
        pltpu.sync_copy(
            acc.at[pl.ds(ss * ROWS_PT, ROWS_PT)],
            out.at[cc * NCHUNK + p, pl.ds(ss * ROWS_PT, ROWS_PT)])
        if p < NCHUNK - 1:
            _zero_my_rows()


def _sc_segsum(h_flat, src_all, dst_all, zrows):
    return pl.kernel(
        _sc_body,
        out_type=jax.ShapeDtypeStruct((2 * NCHUNK, NP, 128), jnp.float32),
        mesh=plsc.VectorSubcoreMesh(core_axis_name="c", subcore_axis_name="s"),
        scratch_types=[
            pltpu.VMEM((EPT,), jnp.int32),
            pltpu.VMEM((NU, UR), jnp.int32),
            pltpu.VMEM((RING, UR, 128), jnp.float32),
            pltpu.VMEM_SHARED((NACC, 128), jnp.float32),
            pltpu.SemaphoreType.DMA((RING,)),
            pltpu.SemaphoreType.DMA((RING,)),
        ],
    )(h_flat, src_all, dst_all, zrows)


def _mlp_body(eps_ref, h_ref, a1_ref, a2_ref, w1a, b1a, w1b, b1b, w2a, b2a,
              w2b, b2b, m1a, m1b, bm1, m2, bm2, out_ref, *, relu_out):
    f32 = jnp.float32
    e1 = eps_ref[0, 0]
    e2 = eps_ref[0, 1]
    h = jnp.concatenate([h_ref[c] for c in range(NCHUNK)], axis=1)
    a1 = jnp.concatenate([a1_ref[c] for c in range(NCHUNK)], axis=1)
    a2 = jnp.concatenate([a2_ref[c] for c in range(NCHUNK)], axis=1)
    x1 = (1.0 + e1) * h + a1
    x2 = (1.0 + e2) * h + a2
    y1 = jnp.maximum(
        jnp.dot(x1, w1a[...], preferred_element_type=f32) + b1a[...], 0.0)
    t1 = jnp.dot(y1, w1b[...], preferred_element_type=f32) + b1b[...]
    y2 = jnp.maximum(
        jnp.dot(x2, w2a[...], preferred_element_type=f32) + b2a[...], 0.0)
    t2 = jnp.dot(y2, w2b[...], preferred_element_type=f32) + b2b[...]
    u = jnp.maximum(
        jnp.dot(t1, m1a[...], preferred_element_type=f32)
        + jnp.dot(t2, m1b[...], preferred_element_type=f32) + bm1[...], 0.0)
    z = jnp.dot(u, m2[...], preferred_element_type=f32) + bm2[...]
    if relu_out:
        z = jnp.maximum(z, 0.0)
    for c in range(NCHUNK):
        out_ref[c] = z[:, c * 128:(c + 1) * 128]


def _full(shape):
    return pl.BlockSpec(shape, lambda i: tuple(0 for _ in shape))


def _mlp_layer(eps, h_c, agg, lw, relu_out):
    nblk = NP // BM
    spec_h = pl.BlockSpec((NCHUNK, BM, 128), lambda i: (0, i, 0))
    spec_a1 = pl.BlockSpec((NCHUNK, BM, 128), lambda i: (0, i, 0))
    spec_a2 = pl.BlockSpec((NCHUNK, BM, 128), lambda i: (1, i, 0))
    return pl.pallas_call(
        functools.partial(_mlp_body, relu_out=relu_out),
        grid=(nblk,),
        in_specs=[
            pl.BlockSpec(memory_space=pltpu.SMEM),
            spec_h, spec_a1, spec_a2,
            _full((DP, D2P)), _full((1, D2P)), _full((D2P, DP)),
            _full((1, DP)),
            _full((DP, D2P)), _full((1, D2P)), _full((D2P, DP)),
            _full((1, DP)),
            _full((DP, DP)), _full((DP, DP)), _full((1, DP)),
            _full((DP, DP)), _full((1, DP)),
        ],
        out_specs=pl.BlockSpec((NCHUNK, BM, 128), lambda i: (0, i, 0)),
        out_shape=jax.ShapeDtypeStruct((NCHUNK, NP, 128), jnp.float32),
    )(eps, h_c, agg, agg, *lw)


def _pool_body(h_ref, b_ref, p1, p1b, p2, p2b, out_ref, acc, cnt):
    i = pl.program_id(0)
    f32 = jnp.float32

    @pl.when(i == 0)
    def _():
        acc[...] = jnp.zeros((G, DP), f32)
        cnt[...] = jnp.zeros((G, 128), f32)

    hcat = jnp.concatenate([h_ref[c] for c in range(NCHUNK)], axis=1)
    ids = b_ref[0]
    iot = lax.broadcasted_iota(jnp.int32, (G, 1), 0)
    ones = jnp.ones((128, 128), f32)
    for r in range(8):
        oh = (ids[r:r + 1, :] == iot).astype(f32)
        acc[...] += jnp.dot(oh, hcat[r * 128:(r + 1) * 128, :],
                            preferred_element_type=f32)
        cnt[...] += jnp.dot(oh, ones, preferred_element_type=f32)

    @pl.when(i == NP // BMP - 1)
    def _():
        mean = acc[...] / jnp.maximum(cnt[...][:, 0:1], 1.0)
        q = jnp.maximum(
            jnp.dot(mean, p1[...], preferred_element_type=f32) + p1b[...],
            0.0)
        out_ref[...] = jnp.dot(q, p2[...], preferred_element_type=f32) \
            + p2b[...]


def _pool(h_c, batch_r, p1, p1b, p2, p2b):
    nblk = NP // BMP
    return pl.pallas_call(
        _pool_body,
        grid=(nblk,),
        in_specs=[
            pl.BlockSpec((NCHUNK, BMP, 128), lambda i: (0, i, 0)),
            pl.BlockSpec((1, 8, 128), lambda i: (i, 0, 0)),
            _full((DP, DP)), _full((1, DP)), _full((DP, NT)),
            _full((1, NT)),
        ],
        out_specs=pl.BlockSpec((G, NT), lambda i: (0, 0)),
        out_shape=jax.ShapeDtypeStruct((G, NT), jnp.float32),
        scratch_shapes=[
            pltpu.VMEM((G, DP), jnp.float32),
            pltpu.VMEM((G, 128), jnp.float32),
        ],
    )(h_c, batch_r, p1, p1b, p2, p2b)


def _pad2(a, r, c):
    return jnp.pad(a, ((0, r - a.shape[0]), (0, c - a.shape[1])))


def _pad1(a, n):
    return jnp.pad(a, (0, n - a.shape[0])).reshape(1, n)


_BN_S = 1.0 / math.sqrt(1.0 + 1e-5)


def _prep_conv(cp):
    sg = cp["g1"] * _BN_S
    w1 = _pad2(cp["l1"]["W"] * sg[None, :], DP, D2P)
    b1 = _pad1(cp["l1"]["b"] * sg + cp["be1"], D2P)
    w2 = _pad2(cp["l2"]["W"], D2P, DP)
    b2 = _pad1(cp["l2"]["b"], DP)
    return w1, b1, w2, b2


def kernel(x, edge_index_1, edge_index_2, batch, params):
    d = x.shape[1]

    xp = jnp.pad(x, ((0, NP - x.shape[0]), (0, DP - d)))
    h = xp.reshape(NP, NCHUNK, 128).transpose(1, 0, 2)

    def _edges(ei):
        src = jnp.pad(ei[0], (0, EP - ei.shape[1]))
        dst = jnp.pad(ei[1], (0, EP - ei.shape[1]), constant_values=NNODE)
        return src, dst

    s1, d1 = _edges(edge_index_1)
    s2, d2 = _edges(edge_index_2)
    src_all = jnp.stack([s1, s2]).reshape(32, EPT)
    dst_all = jnp.stack([d1, d2]).reshape(32, NU, UR)
    zrows = jnp.zeros((ROWS_PT, 128), jnp.float32)

    batch_r = jnp.pad(batch, (0, NP - batch.shape[0]),
                      constant_values=G).reshape(NP // BMP, 8, 128)

    sbn = params["bn1_g"] * _BN_S
    bbn = params["bn1_b"]

    nlayer = len(params["layers"])
    for i, lp in enumerate(params["layers"]):
        w1a, b1a, w1b, b1b = _prep_conv(lp["c1"])
        w2a, b2a, w2b, b2b = _prep_conv(lp["c2"])
        m1w = lp["m1"]["W"]
        m1a = _pad2(m1w[:d], DP, DP)
        m1b = _pad2(m1w[d:], DP, DP)
        bm1 = _pad1(lp["m1"]["b"], DP)
        m2 = _pad2(lp["m2"]["W"] * sbn[None, :], DP, DP)
        bm2 = _pad1(lp["m2"]["b"] * sbn + bbn, DP)
        eps = jnp.stack([lp["c1"]["eps"], lp["c2"]["eps"]]).reshape(1, 2)
        lw = (w1a, b1a, w1b, b1b, w2a, b2a, w2b, b2b, m1a, m1b, bm1, m2, bm2)

        agg = _sc_segsum(h.reshape(NCHUNK * NP, 128), src_all, dst_all,
                         zrows)
        h = _mlp_layer(eps, h, agg, lw, relu_out=(i < nlayer - 1))

    p1 = _pad2(params["p1"]["W"], DP, DP)
    p1b = _pad1(params["p1"]["b"], DP)
    p2 = _pad2(params["p2"]["W"], DP, NT)
    p2b = _pad1(params["p2"]["b"], NT)
    return _pool(h, batch_r, p1, p1b, p2, p2b)


# speedup vs baseline: 1.1163x; 1.1163x over previous
import functools
import math

import jax
import jax.numpy as jnp
from jax import lax
from jax.experimental import pallas as pl
from jax.experimental.pallas import tpu as pltpu
from jax.experimental.pallas import tpu_sc as plsc

NNODE = 10000
NP = 10240
DP = 768
NCHUNK = 6
D2P = 1536
EB = 112
NB = 56
EPT = EB * NB
EP = 16 * EPT
NACC = 10112
ROWS_PT = NACC // 16
UR = 56
NU = EPT // UR
RING = 4
ASC = 2
G = 64
NT = 128
BM = 512
BMP = 1024


def _sc_body(h_flat, src_all, dst_all, zrows, out, src_v, dst_v, gbuf,
             acc, gsem, ssem, wsem):
    cc = lax.axis_index("c")
    ss = lax.axis_index("s")
    w = cc * 16 + ss

    pltpu.sync_copy(src_all.at[w], src_v)
    pltpu.sync_copy(dst_all.at[w], dst_v)

    def _zero_my_rows():
        pltpu.sync_copy(zrows, acc.at[pl.ds(ss * ROWS_PT, ROWS_PT)])

    def _gather(b, sel):
        pltpu.async_copy(h_flat.at[src_v.at[pl.ds(b * UR, UR)]],
                         gbuf.at[sel], gsem.at[sel])

    def _gather_wait(b, sel):
        pltpu.make_async_copy(h_flat.at[src_v.at[pl.ds(b * UR, UR)]],
                              gbuf.at[sel], gsem.at[sel]).wait()

    def _scat(b, sel):
        pltpu.async_copy(gbuf.at[sel], acc.at[dst_v.at[b]], ssem.at[sel],
                         add=True)

    def _scat_wait(b, sel):
        pltpu.make_async_copy(gbuf.at[sel], acc.at[dst_v.at[b]],
                              ssem.at[sel]).wait()

    def _advance_src():
        def ob(j, carry):
            src_v[pl.ds(j * 16, 16)] = src_v[pl.ds(j * 16, 16)] + NP
            return carry
        lax.fori_loop(0, EPT // 16, ob, 0)

    _zero_my_rows()
    plsc.subcore_barrier()
    for i in range(RING):
        _gather(i, i)

    for p in range(NCHUNK):
        def eb(g, carry):
            for r in range(RING):
                b = g * RING + r
                fs = (r - ASC) % RING

                @pl.when(b >= ASC)
                def _():
                    _scat_wait(b - ASC, fs)

                    @pl.when(b - ASC + RING < NU)
                    def _():
                        _gather(b - ASC + RING, fs)
                _gather_wait(b, r)
                _scat(b, r)
            return carry
        lax.fori_loop(0, NU // RING, eb, 0)
        for k in range(ASC):
            b = NU - ASC + k
            _scat_wait(b, b % RING)

        plsc.subcore_barrier()
        pltpu.async_copy(
            acc.at[pl.ds(ss * ROWS_PT, ROWS_PT)],
            out.at[cc * NCHUNK + p, pl.ds(ss * ROWS_PT, ROWS_PT)], wsem)
        if p < NCHUNK - 1:
            _advance_src()
            for i in range(RING):
                _gather(i, i)
        pltpu.make_async_copy(
            acc.at[pl.ds(ss * ROWS_PT, ROWS_PT)],
            out.at[cc * NCHUNK + p, pl.ds(ss * ROWS_PT, ROWS_PT)],
            wsem).wait()
        if p < NCHUNK - 1:
            _zero_my_rows()
            plsc.subcore_barrier()


def _sc_segsum(h_flat, src_all, dst_all, zrows):
    return pl.kernel(
        _sc_body,
        out_type=jax.ShapeDtypeStruct((2 * NCHUNK, NP, 128), jnp.float32),
        mesh=plsc.VectorSubcoreMesh(core_axis_name="c", subcore_axis_name="s"),
        scratch_types=[
            pltpu.VMEM((EPT,), jnp.int32),
            pltpu.VMEM((NU, UR), jnp.int32),
            pltpu.VMEM((RING, UR, 128), jnp.float32),
            pltpu.VMEM_SHARED((NACC, 128), jnp.float32),
            pltpu.SemaphoreType.DMA((RING,)),
            pltpu.SemaphoreType.DMA((RING,)),
            pltpu.SemaphoreType.DMA,
        ],
    )(h_flat, src_all, dst_all, zrows)


def _mlp_body(eps_ref, h_ref, a1_ref, a2_ref, w1a, b1a, w1b, b1b, w2a, b2a,
              w2b, b2b, m1a, m1b, bm1, m2, bm2, out_ref, *, relu_out):
    f32 = jnp.float32
    e1 = eps_ref[0, 0]
    e2 = eps_ref[0, 1]
    h = jnp.concatenate([h_ref[c] for c in range(NCHUNK)], axis=1)
    a1 = jnp.concatenate([a1_ref[c] for c in range(NCHUNK)], axis=1)
    a2 = jnp.concatenate([a2_ref[c] for c in range(NCHUNK)], axis=1)
    x1 = (1.0 + e1) * h + a1
    x2 = (1.0 + e2) * h + a2
    y1 = jnp.maximum(
        jnp.dot(x1, w1a[...], preferred_element_type=f32) + b1a[...], 0.0)
    t1 = jnp.dot(y1, w1b[...], preferred_element_type=f32) + b1b[...]
    y2 = jnp.maximum(
        jnp.dot(x2, w2a[...], preferred_element_type=f32) + b2a[...], 0.0)
    t2 = jnp.dot(y2, w2b[...], preferred_element_type=f32) + b2b[...]
    u = jnp.maximum(
        jnp.dot(t1, m1a[...], preferred_element_type=f32)
        + jnp.dot(t2, m1b[...], preferred_element_type=f32) + bm1[...], 0.0)
    z = jnp.dot(u, m2[...], preferred_element_type=f32) + bm2[...]
    if relu_out:
        z = jnp.maximum(z, 0.0)
    for c in range(NCHUNK):
        out_ref[c] = z[:, c * 128:(c + 1) * 128]


def _full(shape):
    return pl.BlockSpec(shape, lambda i: tuple(0 for _ in shape))


def _mlp_layer(eps, h_c, agg, lw, relu_out):
    nblk = NP // BM
    spec_h = pl.BlockSpec((NCHUNK, BM, 128), lambda i: (0, i, 0))
    spec_a1 = pl.BlockSpec((NCHUNK, BM, 128), lambda i: (0, i, 0))
    spec_a2 = pl.BlockSpec((NCHUNK, BM, 128), lambda i: (1, i, 0))
    return pl.pallas_call(
        functools.partial(_mlp_body, relu_out=relu_out),
        grid=(nblk,),
        in_specs=[
            pl.BlockSpec(memory_space=pltpu.SMEM),
            spec_h, spec_a1, spec_a2,
            _full((DP, D2P)), _full((1, D2P)), _full((D2P, DP)),
            _full((1, DP)),
            _full((DP, D2P)), _full((1, D2P)), _full((D2P, DP)),
            _full((1, DP)),
            _full((DP, DP)), _full((DP, DP)), _full((1, DP)),
            _full((DP, DP)), _full((1, DP)),
        ],
        out_specs=pl.BlockSpec((NCHUNK, BM, 128), lambda i: (0, i, 0)),
        out_shape=jax.ShapeDtypeStruct((NCHUNK, NP, 128), jnp.float32),
    )(eps, h_c, agg, agg, *lw)


def _pool_body(h_ref, b_ref, p1, p1b, p2, p2b, out_ref, acc, cnt):
    i = pl.program_id(0)
    f32 = jnp.float32

    @pl.when(i == 0)
    def _():
        acc[...] = jnp.zeros((G, DP), f32)
        cnt[...] = jnp.zeros((G, 128), f32)

    hcat = jnp.concatenate([h_ref[c] for c in range(NCHUNK)], axis=1)
    ids = b_ref[0]
    iot = lax.broadcasted_iota(jnp.int32, (G, 1), 0)
    ones = jnp.ones((128, 128), f32)
    for r in range(8):
        oh = (ids[r:r + 1, :] == iot).astype(f32)
        acc[...] += jnp.dot(oh, hcat[r * 128:(r + 1) * 128, :],
                            preferred_element_type=f32)
        cnt[...] += jnp.dot(oh, ones, preferred_element_type=f32)

    @pl.when(i == NP // BMP - 1)
    def _():
        mean = acc[...] / jnp.maximum(cnt[...][:, 0:1], 1.0)
        q = jnp.maximum(
            jnp.dot(mean, p1[...], preferred_element_type=f32) + p1b[...],
            0.0)
        out_ref[...] = jnp.dot(q, p2[...], preferred_element_type=f32) \
            + p2b[...]


def _pool(h_c, batch_r, p1, p1b, p2, p2b):
    nblk = NP // BMP
    return pl.pallas_call(
        _pool_body,
        grid=(nblk,),
        in_specs=[
            pl.BlockSpec((NCHUNK, BMP, 128), lambda i: (0, i, 0)),
            pl.BlockSpec((1, 8, 128), lambda i: (i, 0, 0)),
            _full((DP, DP)), _full((1, DP)), _full((DP, NT)),
            _full((1, NT)),
        ],
        out_specs=pl.BlockSpec((G, NT), lambda i: (0, 0)),
        out_shape=jax.ShapeDtypeStruct((G, NT), jnp.float32),
        scratch_shapes=[
            pltpu.VMEM((G, DP), jnp.float32),
            pltpu.VMEM((G, 128), jnp.float32),
        ],
    )(h_c, batch_r, p1, p1b, p2, p2b)


def _pad2(a, r, c):
    return jnp.pad(a, ((0, r - a.shape[0]), (0, c - a.shape[1])))


def _pad1(a, n):
    return jnp.pad(a, (0, n - a.shape[0])).reshape(1, n)


_BN_S = 1.0 / math.sqrt(1.0 + 1e-5)


def _prep_conv(cp):
    sg = cp["g1"] * _BN_S
    w1 = _pad2(cp["l1"]["W"] * sg[None, :], DP, D2P)
    b1 = _pad1(cp["l1"]["b"] * sg + cp["be1"], D2P)
    w2 = _pad2(cp["l2"]["W"], D2P, DP)
    b2 = _pad1(cp["l2"]["b"], DP)
    return w1, b1, w2, b2


def kernel(x, edge_index_1, edge_index_2, batch, params):
    d = x.shape[1]

    xp = jnp.pad(x, ((0, NP - x.shape[0]), (0, DP - d)))
    h = xp.reshape(NP, NCHUNK, 128).transpose(1, 0, 2)

    def _edges(ei):
        src = jnp.pad(ei[0], (0, EP - ei.shape[1]))
        dst = jnp.pad(ei[1], (0, EP - ei.shape[1]), constant_values=NNODE)
        return src, dst

    s1, d1 = _edges(edge_index_1)
    s2, d2 = _edges(edge_index_2)
    src_all = jnp.stack([s1, s2]).reshape(32, EPT)
    dst_all = jnp.stack([d1, d2]).reshape(32, NU, UR)
    zrows = jnp.zeros((ROWS_PT, 128), jnp.float32)

    batch_r = jnp.pad(batch, (0, NP - batch.shape[0]),
                      constant_values=G).reshape(NP // BMP, 8, 128)

    sbn = params["bn1_g"] * _BN_S
    bbn = params["bn1_b"]

    nlayer = len(params["layers"])
    for i, lp in enumerate(params["layers"]):
        w1a, b1a, w1b, b1b = _prep_conv(lp["c1"])
        w2a, b2a, w2b, b2b = _prep_conv(lp["c2"])
        m1w = lp["m1"]["W"]
        m1a = _pad2(m1w[:d], DP, DP)
        m1b = _pad2(m1w[d:], DP, DP)
        bm1 = _pad1(lp["m1"]["b"], DP)
        m2 = _pad2(lp["m2"]["W"] * sbn[None, :], DP, DP)
        bm2 = _pad1(lp["m2"]["b"] * sbn + bbn, DP)
        eps = jnp.stack([lp["c1"]["eps"], lp["c2"]["eps"]]).reshape(1, 2)
        lw = (w1a, b1a, w1b, b1b, w2a, b2a, w2b, b2b, m1a, m1b, bm1, m2, bm2)

        agg = _sc_segsum(h.reshape(NCHUNK * NP, 128), src_all, dst_all,
                         zrows)
        h = _mlp_layer(eps, h, agg, lw, relu_out=(i < nlayer - 1))

    p1 = _pad2(params["p1"]["W"], DP, DP)
    p1b = _pad1(params["p1"]["b"], DP)
    p2 = _pad2(params["p2"]["W"], DP, NT)
    p2b = _pad1(params["p2"]["b"], NT)
    return _pool(h, batch_r, p1, p1b, p2, p2b)
